# Initial kernel scaffold; baseline (speedup 1.0000x reference)
import jax
import jax.numpy as jnp
from jax.experimental import pallas as pl


def kernel(x, params, edge_index, batch):
    raise NotImplementedError("write your pallas kernel here")


# speedup vs baseline: 5.1474x; 5.1474x over previous
import functools

import jax
import jax.numpy as jnp
from jax import lax
from jax.experimental import pallas as pl
from jax.experimental.pallas import tpu as pltpu
from jax.experimental.pallas import tpu_sc as plsc

N = 10000
E = 320000
H = 128
NG = 64
COUT = 10
LAYERS = 3

NC = 2
NS = 16
NW = NC * NS
NPAD = 10240
RPW = NPAD // NS
CHUNK = 128
EPW = 10112
NCHUNKS = EPW // CHUNK
EPAD = EPW * NW

RB = 1024
GRID = NPAD // RB

_mesh = plsc.VectorSubcoreMesh(core_axis_name="c", subcore_axis_name="s")


@functools.partial(
    pl.kernel,
    out_type=jax.ShapeDtypeStruct((NC, NPAD, H), jnp.float32),
    mesh=_mesh,
    scratch_types=[
        pltpu.VMEM((CHUNK,), jnp.int32),
        pltpu.VMEM((CHUNK,), jnp.int32),
        pltpu.VMEM((CHUNK, H), jnp.float32),
        pltpu.VMEM_SHARED((NPAD, H), jnp.float32),
        pltpu.SemaphoreType.DMA,
    ],
)
def _edge_sum_wide(x_hbm, src_hbm, dst_hbm, out_hbm, sidx, didx, rows, acc, sem):
    c = lax.axis_index("c")
    s = lax.axis_index("s")
    wid = s * NC + c

    zero = jnp.zeros((16,), jnp.float32)

    def zb(r, carry):
        for j in range(H // 16):
            rows[r, pl.ds(j * 16, 16)] = zero
        return carry

    lax.fori_loop(0, CHUNK, zb, 0)
    rbase = s * RPW
    for j in range(RPW // CHUNK):
        pltpu.sync_copy(rows, acc.at[pl.ds(rbase + j * CHUNK, CHUNK)])
    plsc.subcore_barrier()

    ebase = wid * EPW

    def body(i, carry):
        b = ebase + i * CHUNK
        pltpu.sync_copy(src_hbm.at[pl.ds(b, CHUNK)], sidx)
        pltpu.sync_copy(dst_hbm.at[pl.ds(b, CHUNK)], didx)
        pltpu.async_copy(x_hbm.at[sidx], rows, sem).wait()
        pltpu.sync_copy(rows, acc.at[didx], add=True)
        return carry

    lax.fori_loop(0, NCHUNKS, body, 0)
    plsc.subcore_barrier()
    pltpu.sync_copy(acc.at[pl.ds(rbase, RPW)], out_hbm.at[c, pl.ds(rbase, RPW)])


@functools.partial(
    pl.kernel,
    out_type=jax.ShapeDtypeStruct((NC, NPAD), jnp.float32),
    mesh=_mesh,
    scratch_types=[
        pltpu.VMEM((NPAD,), jnp.float32),
        pltpu.VMEM((NPAD,), jnp.float32),
        pltpu.VMEM((EPW,), jnp.int32),
        pltpu.VMEM((EPW,), jnp.int32),
        pltpu.VMEM((RPW,), jnp.float32),
        pltpu.VMEM_SHARED((NS, NPAD), jnp.float32),
    ],
    compiler_params=pltpu.CompilerParams(needs_layout_passes=False),
)
def _edge_sum_scalar(t_hbm, src_hbm, dst_hbm, out_hbm, tv, acc, sb, db, tmp, shacc):
    c = lax.axis_index("c")
    s = lax.axis_index("s")
    wid = s * NC + c

    pltpu.sync_copy(t_hbm, tv)
    zero = jnp.zeros((16,), jnp.float32)

    def z(i, carry):
        acc[pl.ds(i * 16, 16)] = zero
        return carry

    lax.fori_loop(0, NPAD // 16, z, 0)
    pltpu.sync_copy(src_hbm.at[pl.ds(wid * EPW, EPW)], sb)
    pltpu.sync_copy(dst_hbm.at[pl.ds(wid * EPW, EPW)], db)

    def body(i, carry):
        sv = sb[pl.ds(i * 16, 16)]
        dv = db[pl.ds(i * 16, 16)]
        vals = plsc.load_gather(tv, [sv])
        plsc.addupdate_scatter(acc, [dv], vals)
        return carry

    lax.fori_loop(0, EPW // 16, body, 0)

    pltpu.sync_copy(acc, shacc.at[s])
    plsc.subcore_barrier()

    rbase = s * RPW
    pltpu.sync_copy(shacc.at[0, pl.ds(rbase, RPW)], acc.at[pl.ds(0, RPW)])
    for j in range(1, NS):
        pltpu.sync_copy(shacc.at[j, pl.ds(rbase, RPW)], tmp)

        def addk(k, carry):
            acc[pl.ds(k * 16, 16)] = acc[pl.ds(k * 16, 16)] + tmp[pl.ds(k * 16, 16)]
            return carry

        lax.fori_loop(0, RPW // 16, addk, 0)
    pltpu.sync_copy(acc.at[pl.ds(0, RPW)], out_hbm.at[c, pl.ds(rbase, RPW)])


def _lin0_body(x_ref, w_ref, b_ref, o_ref):
    o_ref[...] = (
        jnp.dot(x_ref[...], w_ref[...], preferred_element_type=jnp.float32)
        + b_ref[...]
    )


def _tc_lin0(x, w, b):
    return pl.pallas_call(
        _lin0_body,
        grid=(GRID,),
        in_specs=[
            pl.BlockSpec((RB, H), lambda i: (i, 0)),
            pl.BlockSpec((H, H), lambda i: (0, 0)),
            pl.BlockSpec((1, H), lambda i: (0, 0)),
        ],
        out_specs=pl.BlockSpec((RB, H), lambda i: (i, 0)),
        out_shape=jax.ShapeDtypeStruct((NPAD, H), jnp.float32),
    )(x, w, b)


def _tca_body(x_ref, p_ref, deg_ref, a_ref, b_ref, ca_ref, uv_ref, cuv_ref, t_ref):
    invd = 1.0 / jnp.maximum(deg_ref[0] + deg_ref[1], 1.0)
    m = (p_ref[0] + p_ref[1]) * invd[:, None]
    z = (
        jnp.dot(x_ref[...], a_ref[...], preferred_element_type=jnp.float32)
        + jnp.dot(m, b_ref[...], preferred_element_type=jnp.float32)
        + ca_ref[...]
    )
    sig = jax.nn.sigmoid(z)
    t_ref[...] = (
        jnp.dot(sig, uv_ref[...], preferred_element_type=jnp.float32) + cuv_ref[...]
    )


def _tc_a(x, p, degp, a, b, ca, uv, cuv):
    return pl.pallas_call(
        _tca_body,
        grid=(GRID,),
        in_specs=[
            pl.BlockSpec((RB, H), lambda i: (i, 0)),
            pl.BlockSpec((NC, RB, H), lambda i: (0, i, 0)),
            pl.BlockSpec((NC, RB), lambda i: (0, i)),
            pl.BlockSpec((H, H), lambda i: (0, 0)),
            pl.BlockSpec((H, H), lambda i: (0, 0)),
            pl.BlockSpec((1, H), lambda i: (0, 0)),
            pl.BlockSpec((H, H), lambda i: (0, 0)),
            pl.BlockSpec((1, H), lambda i: (0, 0)),
        ],
        out_specs=pl.BlockSpec((RB, H), lambda i: (i, 0)),
        out_shape=jax.ShapeDtypeStruct((NPAD, H), jnp.float32),
    )(x, p, degp, a, b, ca, uv, cuv)


def _tcb_body(x_ref, t1_ref, s2_ref, deg_ref, w_ref, bc_ref, xm_ref, xw_ref):
    invd = 1.0 / jnp.maximum(deg_ref[0] + deg_ref[1], 1.0)
    s2 = (s2_ref[0] + s2_ref[1]) * invd
    mv = jax.nn.sigmoid(t1_ref[...] + s2)
    xm = x_ref[...] * mv[:, None]
    xm_ref[...] = xm
    xw_ref[...] = (
        jnp.dot(xm, w_ref[...], preferred_element_type=jnp.float32) + bc_ref[...]
    )


def _tc_b(x, t1, s2p, degp, w2c, bc):
    return pl.pallas_call(
        _tcb_body,
        grid=(GRID,),
        in_specs=[
            pl.BlockSpec((RB, H), lambda i: (i, 0)),
            pl.BlockSpec((RB,), lambda i: (i,)),
            pl.BlockSpec((NC, RB), lambda i: (0, i)),
            pl.BlockSpec((NC, RB), lambda i: (0, i)),
            pl.BlockSpec((H, H), lambda i: (0, 0)),
            pl.BlockSpec((1, H), lambda i: (0, 0)),
        ],
        out_specs=[
            pl.BlockSpec((RB, H), lambda i: (i, 0)),
            pl.BlockSpec((RB, H), lambda i: (i, 0)),
        ],
        out_shape=[
            jax.ShapeDtypeStruct((NPAD, H), jnp.float32),
            jax.ShapeDtypeStruct((NPAD, H), jnp.float32),
        ],
    )(x, t1, s2p, degp, w2c, bc)


def _tcc_body(q_ref, xw_ref, w1_ref, batch_ref, xn_ref, pool_ref):
    i = pl.program_id(0)
    aggr = q_ref[0] + q_ref[1]
    xn = jnp.maximum(
        jnp.dot(aggr, w1_ref[...], preferred_element_type=jnp.float32) + xw_ref[...],
        0.0,
    )
    xn_ref[...] = xn
    bb = batch_ref[...]
    oh = (
        bb[None, :] == lax.broadcasted_iota(jnp.int32, (NG, RB), 0)
    ).astype(jnp.float32)
    part = jnp.dot(oh, xn, preferred_element_type=jnp.float32)

    @pl.when(i == 0)
    def _():
        pool_ref[...] = part

    @pl.when(i > 0)
    def _():
        pool_ref[...] += part


def _tc_c(q, xw, w1c, batch_pad):
    return pl.pallas_call(
        _tcc_body,
        grid=(GRID,),
        in_specs=[
            pl.BlockSpec((NC, RB, H), lambda i: (0, i, 0)),
            pl.BlockSpec((RB, H), lambda i: (i, 0)),
            pl.BlockSpec((H, H), lambda i: (0, 0)),
            pl.BlockSpec((RB,), lambda i: (i,)),
        ],
        out_specs=[
            pl.BlockSpec((RB, H), lambda i: (i, 0)),
            pl.BlockSpec((NG, H), lambda i: (0, 0)),
        ],
        out_shape=[
            jax.ShapeDtypeStruct((NPAD, H), jnp.float32),
            jax.ShapeDtypeStruct((NG, H), jnp.float32),
        ],
    )(q, xw, w1c, batch_pad)


def _tcf_body(x1_ref, x2_ref, x3_ref, w1_ref, b1_ref, w2_ref, b2_ref, o_ref):
    h = jnp.concatenate([x1_ref[...], x2_ref[...], x3_ref[...]], axis=1)
    h = jnp.maximum(
        jnp.dot(h, w1_ref[...], preferred_element_type=jnp.float32) + b1_ref[...],
        0.0,
    )
    h = jnp.dot(h, w2_ref[...], preferred_element_type=jnp.float32) + b2_ref[...]
    mx = jnp.max(h, axis=1, keepdims=True)
    z = h - mx
    o_ref[...] = z - jnp.log(jnp.sum(jnp.exp(z), axis=1, keepdims=True))


def _tc_final(x1, x2, x3, w1, b1, w2, b2):
    return pl.pallas_call(
        _tcf_body,
        out_shape=jax.ShapeDtypeStruct((NG, COUT), jnp.float32),
    )(x1, x2, x3, w1, b1, w2, b2)


def kernel(x, params, edge_index, batch):
    f32 = jnp.float32
    src = edge_index[0].astype(jnp.int32)
    dst = edge_index[1].astype(jnp.int32)
    pad_e = EPAD - E
    src_p = jnp.concatenate([src, jnp.zeros((pad_e,), jnp.int32)])
    dst_p = jnp.concatenate([dst, jnp.full((pad_e,), NPAD - 1, jnp.int32)])
    x_pad = jnp.concatenate([x, jnp.zeros((NPAD - N, x.shape[1]), f32)])
    batch_pad = jnp.concatenate(
        [batch.astype(jnp.int32), jnp.full((NPAD - N,), NG, jnp.int32)]
    )
    ones = jnp.ones((NPAD,), f32)

    folds = []
    for i in range(LAYERS):
        p1 = params["ma1h"][i]
        p2 = params["ma2h"][i]
        w3 = p1["lin3"]["w"]
        a = p1["lin2"]["w"] @ w3[:H]
        b = p1["lin1"]["w"] @ w3[H:]
        ca = (p1["lin2"]["b"] @ w3[:H] + p1["lin1"]["b"] @ w3[H:] + p1["lin3"]["b"])
        w3p = p2["lin3"]["w"]
        u = p2["lin2"]["w"] @ w3p[:H]
        v = p2["lin1"]["w"] @ w3p[H:]
        c2 = (
            p2["lin2"]["b"] @ w3p[:H] + p2["lin1"]["b"] @ w3p[H:] + p2["lin3"]["b"]
        )[0]
        uv = jnp.concatenate([u, v, jnp.zeros((H, H - 2), f32)], axis=1)
        cuv = jnp.zeros((1, H), f32).at[0, 0].set(c2)
        pc = params["conv"][i]
        bc = (pc["lin1"]["b"] + pc["lin2"]["b"])[None, :]
        folds.append(
            dict(a=a, b=b, ca=ca[None, :], uv=uv, cuv=cuv,
                 w1c=pc["lin1"]["w"], w2c=pc["lin2"]["w"], bc=bc)
        )

    degp = _edge_sum_scalar(ones, src_p, dst_p)
    xc = _tc_lin0(x_pad, params["lin0"]["w"], params["lin0"]["b"][None, :])

    pools = []
    for i in range(LAYERS):
        f = folds[i]
        p = _edge_sum_wide(xc, src_p, dst_p)
        t = _tc_a(xc, p, degp, f["a"], f["b"], f["ca"], f["uv"], f["cuv"])
        t1 = t[:, 0]
        t2 = t[:, 1]
        s2p = _edge_sum_scalar(t2, src_p, dst_p)
        xm, xw = _tc_b(xc, t1, s2p, degp, f["w2c"], f["bc"])
        q = _edge_sum_wide(xm, src_p, dst_p)
        xc, pool_i = _tc_c(q, xw, f["w1c"], batch_pad)
        pools.append(pool_i)

    return _tc_final(
        pools[0], pools[1], pools[2],
        params["lin1"]["w"], params["lin1"]["b"][None, :],
        params["lin2"]["w"], params["lin2"]["b"][None, :],
    )
